# baseline (speedup 1.0000x reference)
import jax
import jax.numpy as jnp
from jax.experimental import pallas as pl


def _top2_kernel(x_ref, out_ref):
    x = x_ref[...]
    m1 = jnp.max(x, axis=1, keepdims=True)
    col = jax.lax.broadcasted_iota(jnp.int32, x.shape, 1)
    big = jnp.iinfo(jnp.int32).max
    first_max_col = jnp.min(jnp.where(x == m1, col, big), axis=1, keepdims=True)
    masked = jnp.where(col == first_max_col, -jnp.inf, x)
    m2 = jnp.max(masked, axis=1, keepdims=True)
    out_ref[...] = jnp.concatenate([m1, m2], axis=1)


def kernel(x):
    values = pl.pallas_call(
        _top2_kernel,
        out_shape=jax.ShapeDtypeStruct((x.shape[0], 2), x.dtype),
    )(x)
    return values[0:0, 0:1]
